# Initial kernel scaffold; baseline (speedup 1.0000x reference)
import jax
import jax.numpy as jnp
from jax.experimental import pallas as pl


def kernel(x, edge_index, W_self1, W_neigh1, b1, W_self2, W_neigh2, b2):
    raise NotImplementedError("write your pallas kernel here")


# speedup vs baseline: 3.0338x; 3.0338x over previous
import functools

import jax
import jax.numpy as jnp
from jax import lax
from jax.experimental import pallas as pl
from jax.experimental.pallas import tpu as pltpu
from jax.experimental.pallas import tpu_sc as plsc

_NC = 2
_NS = 16
_NW = _NC * _NS
_C = 128
_G = 8


def _cdiv(a, b):
    return (a + b - 1) // b


def _sc_segsum(table, src2d, dst2d, zagg, n_pad, ch, rpt, edge_split):
    d = table.shape[2]
    mesh = plsc.VectorSubcoreMesh(core_axis_name="c", subcore_axis_name="s")

    @functools.partial(
        pl.kernel,
        mesh=mesh,
        out_type=jax.ShapeDtypeStruct((_NC * n_pad, d), jnp.float32),
        compiler_params=pltpu.CompilerParams(use_tc_tiling_on_sc=False),
        scratch_types=[
            pltpu.VMEM((_G, _C), jnp.int32),
            pltpu.VMEM((_G, _C), jnp.int32),
            pltpu.VMEM((_C, d), jnp.float32),
            pltpu.VMEM_SHARED((n_pad, d), jnp.float32),
            pltpu.SemaphoreType.DMA,
        ],
    )
    def k(tab_hbm, src_hbm, dst_hbm, zagg_hbm,
          agg_out,
          src_v, dst_v, rows_v, agg_sh, sem):
        c = lax.axis_index("c")
        s = lax.axis_index("s")
        r0 = s * rpt
        pltpu.sync_copy(zagg_hbm.at[pl.ds(0, _C)], rows_v)
        for off in range(0, rpt, _C):
            sz = min(_C, rpt - off)
            pltpu.sync_copy(rows_v.at[pl.ds(0, sz)],
                            agg_sh.at[pl.ds(r0 + off, sz)])
        plsc.subcore_barrier()

        if edge_split:
            tile_base = (c * _NS + s) * ch
            tab = tab_hbm.at[0]
        else:
            tile_base = s * ch
            tab = tab_hbm.at[c]

        def group(g, carry):
            base = tile_base + g * _G
            pltpu.sync_copy(src_hbm.at[pl.ds(base, _G)], src_v)
            pltpu.sync_copy(dst_hbm.at[pl.ds(base, _G)], dst_v)
            for j in range(_G):
                pltpu.async_copy(tab.at[src_v.at[j]], rows_v, sem).wait()
                pltpu.sync_copy(rows_v, agg_sh.at[dst_v.at[j]], add=True)
            return carry

        lax.fori_loop(0, ch // _G, group, 0)
        plsc.subcore_barrier()
        o0 = c * n_pad + r0
        for off in range(0, rpt, _C):
            sz = min(_C, rpt - off)
            pltpu.sync_copy(agg_sh.at[pl.ds(r0 + off, sz)],
                            rows_v.at[pl.ds(0, sz)])
            pltpu.sync_copy(rows_v.at[pl.ds(0, sz)],
                            agg_out.at[pl.ds(o0 + off, sz)])

    return k(table, src2d, dst2d, zagg).reshape(_NC, n_pad, d)


def _tc_layer1(x, agg1, W_self, W_neigh, b, blk):
    n, d_in = x.shape
    d_out = W_self.shape[1]
    grid = (n // blk,)

    def body(x_ref, a_ref, ws_ref, wn_ref, b_ref, out_ref, deg_ref):
        a = a_ref[0] + a_ref[1]
        deg = a[:, 128:129]
        inv = 1.0 / jnp.maximum(deg, 1.0)
        hn = a[:, :128] * inv
        h = (jnp.dot(x_ref[...], ws_ref[...], preferred_element_type=jnp.float32)
             + jnp.dot(hn, wn_ref[...], preferred_element_type=jnp.float32)
             + b_ref[...])
        h = jnp.maximum(h, 0.0)
        out_ref[0] = h[:, :128]
        out_ref[1] = h[:, 128:]
        deg_ref[...] = a[:, 128:144]

    return pl.pallas_call(
        body,
        grid=grid,
        in_specs=[
            pl.BlockSpec((blk, d_in), lambda i: (i, 0)),
            pl.BlockSpec((2, blk, 144), lambda i: (0, i, 0)),
            pl.BlockSpec((d_in, d_out), lambda i: (0, 0)),
            pl.BlockSpec((d_in, d_out), lambda i: (0, 0)),
            pl.BlockSpec((1, d_out), lambda i: (0, 0)),
        ],
        out_specs=(
            pl.BlockSpec((2, blk, 128), lambda i: (0, i, 0)),
            pl.BlockSpec((blk, 16), lambda i: (i, 0)),
        ),
        out_shape=(
            jax.ShapeDtypeStruct((2, n, 128), jnp.float32),
            jax.ShapeDtypeStruct((n, 16), jnp.float32),
        ),
    )(x, agg1, W_self, W_neigh, b)


def _tc_layer2(h1h, agg2, deg, W_self, W_neigh, b, blk):
    n = h1h.shape[1]
    d = W_self.shape[0]
    d_out = W_self.shape[1]
    grid = (n // blk,)

    def body(h_ref, a_ref, deg_ref, ws_ref, wn_ref, b_ref, out_ref):
        deg = deg_ref[...][:, 0:1]
        inv = 1.0 / jnp.maximum(deg, 1.0)
        h1 = jnp.concatenate([h_ref[0], h_ref[1]], axis=1)
        hn = jnp.concatenate([a_ref[0], a_ref[1]], axis=1) * inv
        h = (jnp.dot(h1, ws_ref[...], preferred_element_type=jnp.float32)
             + jnp.dot(hn, wn_ref[...], preferred_element_type=jnp.float32)
             + b_ref[...])
        out_ref[...] = jnp.maximum(h, 0.0)

    return pl.pallas_call(
        body,
        grid=grid,
        in_specs=[
            pl.BlockSpec((2, blk, 128), lambda i: (0, i, 0)),
            pl.BlockSpec((2, blk, 128), lambda i: (0, i, 0)),
            pl.BlockSpec((blk, 16), lambda i: (i, 0)),
            pl.BlockSpec((d, d_out), lambda i: (0, 0)),
            pl.BlockSpec((d, d_out), lambda i: (0, 0)),
            pl.BlockSpec((1, d_out), lambda i: (0, 0)),
        ],
        out_specs=pl.BlockSpec((blk, d_out), lambda i: (i, 0)),
        out_shape=jax.ShapeDtypeStruct((n, d_out), jnp.float32),
    )(h1h, agg2, deg, W_self, W_neigh, b)


def kernel(x, edge_index, W_self1, W_neigh1, b1, W_self2, W_neigh2, b2):
    n, d_in = x.shape
    e = edge_index.shape[1]

    ch1 = _cdiv(_cdiv(e, _C * _NW), 8) * 8
    ch_tot = _NW * ch1
    e_pad = ch_tot * _C
    ch2 = ch_tot // _NS
    rpt = _cdiv(_cdiv(n + 1, _NS), 8) * 8
    n_pad = rpt * _NS

    src = edge_index[0].astype(jnp.int32)
    dst = edge_index[1].astype(jnp.int32)
    src2d = jnp.pad(src, (0, e_pad - e)).reshape(ch_tot, _C)
    dst2d = jnp.pad(dst, (0, e_pad - e), constant_values=n).reshape(ch_tot, _C)
    zagg1 = jnp.zeros((n_pad, 144), jnp.float32)
    zagg2 = jnp.zeros((n_pad, 128), jnp.float32)
    x_aug = jnp.concatenate([x, jnp.ones((n, 16), jnp.float32)], axis=1)
    x_aug = x_aug.reshape(1, n, 144)
    b1r = b1.reshape(1, -1)
    b2r = b2.reshape(1, -1)

    agg1 = _sc_segsum(x_aug, src2d, dst2d, zagg1, n_pad, ch1, rpt,
                      edge_split=True)
    h1h, deg = _tc_layer1(x, agg1, W_self1, W_neigh1, b1r, blk=1000)
    agg2 = _sc_segsum(h1h, src2d, dst2d, zagg2, n_pad, ch2, rpt,
                      edge_split=False)
    h2 = _tc_layer2(h1h, agg2, deg, W_self2, W_neigh2, b2r, blk=1000)
    return h2


# speedup vs baseline: 3.3414x; 1.1014x over previous
import functools

import jax
import jax.numpy as jnp
from jax import lax
from jax.experimental import pallas as pl
from jax.experimental.pallas import tpu as pltpu
from jax.experimental.pallas import tpu_sc as plsc

_NC = 2
_NS = 16
_NW = _NC * _NS
_C = 128
_G = 8


def _cdiv(a, b):
    return (a + b - 1) // b


def _sc_segsum(table, src2d, dst2d, zagg, n_pad, ch, rpt, edge_split, grp):
    d = table.shape[2]
    mesh = plsc.VectorSubcoreMesh(core_axis_name="c", subcore_axis_name="s")

    @functools.partial(
        pl.kernel,
        mesh=mesh,
        out_type=jax.ShapeDtypeStruct((_NC * n_pad, d), jnp.float32),
        compiler_params=pltpu.CompilerParams(use_tc_tiling_on_sc=False),
        scratch_types=[
            pltpu.VMEM((grp, _C), jnp.int32),
            pltpu.VMEM((grp, _C), jnp.int32),
            pltpu.VMEM((_C, d), jnp.float32),
            pltpu.VMEM((_C, d), jnp.float32),
            pltpu.VMEM_SHARED((n_pad, d), jnp.float32),
            pltpu.SemaphoreType.DMA,
        ],
    )
    def k(tab_hbm, src_hbm, dst_hbm, zagg_hbm,
          agg_out,
          src_v, dst_v, rows_v, rows_w, agg_sh, sem):
        c = lax.axis_index("c")
        s = lax.axis_index("s")
        r0 = s * rpt
        pltpu.sync_copy(zagg_hbm.at[pl.ds(0, _C)], rows_v)
        for off in range(0, rpt, _C):
            sz = min(_C, rpt - off)
            pltpu.sync_copy(rows_v.at[pl.ds(0, sz)],
                            agg_sh.at[pl.ds(r0 + off, sz)])
        plsc.subcore_barrier()

        if edge_split:
            tile_base = (c * _NS + s) * ch
            tab = tab_hbm.at[0]
        else:
            tile_base = s * ch
            tab = tab_hbm.at[c]

        bufs = [rows_v, rows_w]

        def group(g, carry):
            base = tile_base + g * grp
            pltpu.sync_copy(src_hbm.at[pl.ds(base, grp)], src_v)
            pltpu.sync_copy(dst_hbm.at[pl.ds(base, grp)], dst_v)
            pend = pltpu.async_copy(tab.at[src_v.at[0]], bufs[0], sem)
            for j in range(grp):
                pend.wait()
                if j + 1 < grp:
                    pend = pltpu.async_copy(tab.at[src_v.at[j + 1]],
                                            bufs[(j + 1) % 2], sem)
                pltpu.sync_copy(bufs[j % 2], agg_sh.at[dst_v.at[j]], add=True)
            return carry

        lax.fori_loop(0, ch // grp, group, 0)
        plsc.subcore_barrier()
        o0 = c * n_pad + r0
        for off in range(0, rpt, _C):
            sz = min(_C, rpt - off)
            pltpu.sync_copy(agg_sh.at[pl.ds(r0 + off, sz)],
                            rows_v.at[pl.ds(0, sz)])
            pltpu.sync_copy(rows_v.at[pl.ds(0, sz)],
                            agg_out.at[pl.ds(o0 + off, sz)])

    return k(table, src2d, dst2d, zagg).reshape(_NC, n_pad, d)


def _tc_layer1(x, agg1, W_self, W_neigh, b, blk):
    n, d_in = x.shape
    d_out = W_self.shape[1]
    grid = (n // blk,)

    def body(x_ref, a_ref, ws_ref, wn_ref, b_ref, out_ref, deg_ref):
        a = a_ref[0] + a_ref[1]
        deg = a[:, 128:129]
        inv = 1.0 / jnp.maximum(deg, 1.0)
        hn = a[:, :128] * inv
        h = (jnp.dot(x_ref[...], ws_ref[...], preferred_element_type=jnp.float32)
             + jnp.dot(hn, wn_ref[...], preferred_element_type=jnp.float32)
             + b_ref[...])
        h = jnp.maximum(h, 0.0)
        out_ref[0] = h[:, :128]
        out_ref[1] = h[:, 128:]
        deg_ref[...] = a[:, 128:144]

    return pl.pallas_call(
        body,
        grid=grid,
        in_specs=[
            pl.BlockSpec((blk, d_in), lambda i: (i, 0)),
            pl.BlockSpec((2, blk, 144), lambda i: (0, i, 0)),
            pl.BlockSpec((d_in, d_out), lambda i: (0, 0)),
            pl.BlockSpec((d_in, d_out), lambda i: (0, 0)),
            pl.BlockSpec((1, d_out), lambda i: (0, 0)),
        ],
        out_specs=(
            pl.BlockSpec((2, blk, 128), lambda i: (0, i, 0)),
            pl.BlockSpec((blk, 16), lambda i: (i, 0)),
        ),
        out_shape=(
            jax.ShapeDtypeStruct((2, n, 128), jnp.float32),
            jax.ShapeDtypeStruct((n, 16), jnp.float32),
        ),
    )(x, agg1, W_self, W_neigh, b)


def _tc_layer2(h1h, agg2, deg, W_self, W_neigh, b, blk):
    n = h1h.shape[1]
    d = W_self.shape[0]
    d_out = W_self.shape[1]
    grid = (n // blk,)

    def body(h_ref, a_ref, deg_ref, ws_ref, wn_ref, b_ref, out_ref):
        deg = deg_ref[...][:, 0:1]
        inv = 1.0 / jnp.maximum(deg, 1.0)
        h1 = jnp.concatenate([h_ref[0], h_ref[1]], axis=1)
        hn = jnp.concatenate([a_ref[0], a_ref[1]], axis=1) * inv
        h = (jnp.dot(h1, ws_ref[...], preferred_element_type=jnp.float32)
             + jnp.dot(hn, wn_ref[...], preferred_element_type=jnp.float32)
             + b_ref[...])
        out_ref[...] = jnp.maximum(h, 0.0)

    return pl.pallas_call(
        body,
        grid=grid,
        in_specs=[
            pl.BlockSpec((2, blk, 128), lambda i: (0, i, 0)),
            pl.BlockSpec((2, blk, 128), lambda i: (0, i, 0)),
            pl.BlockSpec((blk, 16), lambda i: (i, 0)),
            pl.BlockSpec((d, d_out), lambda i: (0, 0)),
            pl.BlockSpec((d, d_out), lambda i: (0, 0)),
            pl.BlockSpec((1, d_out), lambda i: (0, 0)),
        ],
        out_specs=pl.BlockSpec((blk, d_out), lambda i: (i, 0)),
        out_shape=jax.ShapeDtypeStruct((n, d_out), jnp.float32),
    )(h1h, agg2, deg, W_self, W_neigh, b)


def kernel(x, edge_index, W_self1, W_neigh1, b1, W_self2, W_neigh2, b2):
    n, d_in = x.shape
    e = edge_index.shape[1]

    ch1 = _cdiv(_cdiv(e, _C * _NW), 8) * 8
    ch_tot = _NW * ch1
    e_pad = ch_tot * _C
    ch2 = ch_tot // _NS
    rpt = _cdiv(_cdiv(n + 1, _NS), 8) * 8
    n_pad = rpt * _NS

    src = edge_index[0].astype(jnp.int32)
    dst = edge_index[1].astype(jnp.int32)
    src2d = jnp.pad(src, (0, e_pad - e)).reshape(ch_tot, _C)
    dst2d = jnp.pad(dst, (0, e_pad - e), constant_values=n).reshape(ch_tot, _C)
    zagg1 = jnp.zeros((n_pad, 144), jnp.float32)
    zagg2 = jnp.zeros((n_pad, 128), jnp.float32)
    x_aug = jnp.concatenate([x, jnp.ones((n, 16), jnp.float32)], axis=1)
    x_aug = x_aug.reshape(1, n, 144)
    b1r = b1.reshape(1, -1)
    b2r = b2.reshape(1, -1)

    agg1 = _sc_segsum(x_aug, src2d, dst2d, zagg1, n_pad, ch1, rpt,
                      edge_split=True, grp=8)
    h1h, deg = _tc_layer1(x, agg1, W_self1, W_neigh1, b1r, blk=1000)
    agg2 = _sc_segsum(h1h, src2d, dst2d, zagg2, n_pad, ch2, rpt,
                      edge_split=False, grp=16)
    h2 = _tc_layer2(h1h, agg2, deg, W_self2, W_neigh2, b2r, blk=1000)
    return h2
